# baseline (speedup 1.0000x reference)
import functools

import jax
import jax.numpy as jnp
from jax import lax
from jax.experimental import pallas as pl
from jax.experimental.pallas import tpu as pltpu
from jax.experimental.pallas import tpu_sc as plsc

N = 10000
E = 320000
D = 128
HC1 = 256
GC = 128
PC1 = 144
C2 = 40
PC2 = 48
NC = 2
NS = 16
LN = 16
B = 64
T1 = 32
T2 = 16
EPT1 = E // NS
EPT2 = E // (NC * NS)
RB = 624
WCH = 16

_f32 = jnp.float32
_i32 = jnp.int32


def _stage_a_body(x_ref, wl_ref, wr_ref, xl_ref, xr_ref):
    xb = x_ref[...]
    ml = jnp.dot(xb, wl_ref[...], preferred_element_type=_f32)
    mr = jnp.dot(xb, wr_ref[...], preferred_element_type=_f32)
    xl_ref[0, :, 0:GC] = ml
    col = lax.broadcasted_iota(_i32, (ml.shape[0], PC1 - GC), 1)
    xl_ref[0, :, GC:PC1] = jnp.where(col < 4, 1.0, 0.0).astype(_f32)
    xr_ref[0, :, :] = mr


def _stage_a(x, W1l, W1r):
    bn = 1000
    grid = (N // bn, 2)
    return pl.pallas_call(
        _stage_a_body,
        grid=grid,
        in_specs=[
            pl.BlockSpec((bn, D), lambda i, g: (i, 0)),
            pl.BlockSpec((D, GC), lambda i, g: (0, g)),
            pl.BlockSpec((D, GC), lambda i, g: (0, g)),
        ],
        out_specs=[
            pl.BlockSpec((1, bn, PC1), lambda i, g: (g, i, 0)),
            pl.BlockSpec((1, bn, GC), lambda i, g: (g, i, 0)),
        ],
        out_shape=[
            jax.ShapeDtypeStruct((2, N, PC1), _f32),
            jax.ShapeDtypeStruct((2, N, GC), _f32),
        ],
    )(x, W1l, W1r)


def _sc_l1_body(xl_hbm, xr_hbm, src_hbm, dst_hbm, att_hbm, out_hbm,
                acc_sh, xl_v0, xr_v0, gs_v0, gd_v0, di_v0, dc_v0,
                xl_v1, xr_v1, gs_v1, gd_v1, di_v1, dc_v1,
                gs_t, gd_t, di_t, att_v, gsem0, gsem1, isem0, isem1):
    c = lax.axis_index("c")
    s = lax.axis_index("s")

    pltpu.sync_copy(att_hbm.at[c], att_v)

    for i in range(WCH):
        for j in range(PC1 // LN):
            xl_v0[i, pl.ds(j * LN, LN)] = jnp.zeros((LN,), _f32)
    rb = s * RB
    nk = jnp.where(s == NS - 1, 40, 39)

    def _zc(k, _):
        pltpu.sync_copy(xl_v0.at[pl.ds(0, WCH)],
                        acc_sh.at[pl.ds(rb + k * WCH, WCH)])
        return 0
    lax.fori_loop(0, nk, _zc, 0)
    plsc.subcore_barrier()

    ebase = s * EPT1
    lanes = lax.broadcasted_iota(_i32, (LN,), 0)
    coff = c * N
    att_regs = [att_v[pl.ds(j * LN, LN)] for j in range(GC // LN)]

    def _stage(base, gs_v, di_v, isem):
        pltpu.async_copy(src_hbm.at[pl.ds(base, B)], gs_v, isem)
        pltpu.async_copy(dst_hbm.at[pl.ds(base, B)], di_v, isem)

    def _fire(base, xl_v, xr_v, gs_v, gd_v, di_v, dc_v, gsem, isem):
        pltpu.make_async_copy(src_hbm.at[pl.ds(base, B)], gs_v, isem).wait()
        pltpu.make_async_copy(dst_hbm.at[pl.ds(base, B)], di_v, isem).wait()
        for i in range(B // LN):
            sl = pl.ds(i * LN, LN)
            d = di_v[sl]
            gs_v[sl] = gs_v[sl] + coff
            gd_v[sl] = d + coff
            dc_v[sl] = d
        pltpu.async_copy(xl_hbm.at[gs_v], xl_v, gsem)
        pltpu.async_copy(xr_hbm.at[gd_v], xr_v, gsem)

    def _compute(xl_v, xr_v, gs_v, gd_v, dc_v, gsem, stage_fn=None):
        pltpu.make_async_copy(xl_hbm.at[gs_v], xl_v, gsem).wait()
        pltpu.make_async_copy(xr_hbm.at[gd_v], xr_v, gsem).wait()
        if stage_fn is not None:
            stage_fn()

        @plsc.parallel_loop(0, B, unroll=8)
        def _erow(e):
            xlr = [xl_v[e, pl.ds(j * LN, LN)] for j in range(GC // LN)]
            xrr = [xr_v[e, pl.ds(j * LN, LN)] for j in range(GC // LN)]
            ws = []
            for h in range(4):
                u = jnp.zeros((LN,), _f32)
                for j in (2 * h, 2 * h + 1):
                    sv = xlr[j] + xrr[j]
                    tv = jnp.maximum(sv, 0.2 * sv)
                    u = u + tv * att_regs[j]
                for sh in (1, 2, 4, 8):
                    u = u + jnp.take(u, lanes ^ sh)
                ws.append(jnp.exp(u))
            for j in range(GC // LN):
                xl_v[e, pl.ds(j * LN, LN)] = xlr[j] * ws[j // 2]
            wp = jnp.where(lanes == 0, ws[0],
                           jnp.where(lanes == 1, ws[1],
                                     jnp.where(lanes == 2, ws[2],
                                               jnp.where(lanes == 3, ws[3],
                                                         0.0))))
            xl_v[e, pl.ds(GC, LN)] = wp

        pltpu.sync_copy(xl_v, acc_sh.at[dc_v], add=True)

    nch = EPT1 // B
    npair = nch // 2
    s0 = (xl_v0, xr_v0, gs_v0, gd_v0, di_v0, dc_v0, gsem0, isem0)
    s1 = (xl_v1, xr_v1, gs_v1, gd_v1, di_v1, dc_v1, gsem1, isem1)

    def _idx(slot):
        return (slot[2], slot[4], slot[7])

    def _cmp(slot):
        return (slot[0], slot[1], slot[2], slot[3], slot[5], slot[6])

    _stage(ebase, *_idx(s0))
    _fire(ebase, *s0)
    _stage(ebase + B, *_idx(s1))
    _fire(ebase + B, *s1)

    def _pair(m, _):
        k0 = 2 * m

        def _st0():
            @pl.when(k0 + 2 < nch)
            def _():
                _stage(ebase + (k0 + 2) * B, *_idx(s0))

        def _st1():
            @pl.when(k0 + 3 < nch)
            def _():
                _stage(ebase + (k0 + 3) * B, *_idx(s1))

        _compute(*_cmp(s0), stage_fn=_st0)

        @pl.when(k0 + 2 < nch)
        def _():
            _fire(ebase + (k0 + 2) * B, *s0)
        _compute(*_cmp(s1), stage_fn=_st1)

        @pl.when(k0 + 3 < nch)
        def _():
            _fire(ebase + (k0 + 3) * B, *s1)
        return 0
    lax.fori_loop(0, npair, _pair, 0)

    tb = ebase + nch * B
    pltpu.sync_copy(src_hbm.at[pl.ds(tb, T1)], gs_t)
    pltpu.sync_copy(dst_hbm.at[pl.ds(tb, T1)], di_t)
    for i in range(T1 // LN):
        sl = pl.ds(i * LN, LN)
        gs_t[sl] = gs_t[sl] + coff
        gd_t[sl] = di_t[sl] + coff
    xl_t = xl_v0.at[pl.ds(0, T1)]
    xr_t = xr_v0.at[pl.ds(0, T1)]
    cp1 = pltpu.async_copy(xl_hbm.at[gs_t], xl_t, gsem0)
    cp2 = pltpu.async_copy(xr_hbm.at[gd_t], xr_t, gsem0)
    cp1.wait()
    cp2.wait()

    @plsc.parallel_loop(0, T1, unroll=8)
    def _erow_t(e):
        xlr = [xl_v0[e, pl.ds(j * LN, LN)] for j in range(GC // LN)]
        xrr = [xr_v0[e, pl.ds(j * LN, LN)] for j in range(GC // LN)]
        ws = []
        for h in range(4):
            u = jnp.zeros((LN,), _f32)
            for j in (2 * h, 2 * h + 1):
                sv = xlr[j] + xrr[j]
                tv = jnp.maximum(sv, 0.2 * sv)
                u = u + tv * att_regs[j]
            for sh in (1, 2, 4, 8):
                u = u + jnp.take(u, lanes ^ sh)
            ws.append(jnp.exp(u))
        for j in range(GC // LN):
            xl_v0[e, pl.ds(j * LN, LN)] = xlr[j] * ws[j // 2]
        wp = jnp.where(lanes == 0, ws[0],
                       jnp.where(lanes == 1, ws[1],
                                 jnp.where(lanes == 2, ws[2],
                                           jnp.where(lanes == 3, ws[3],
                                                     0.0))))
        xl_v0[e, pl.ds(GC, LN)] = wp

    pltpu.sync_copy(xl_t, acc_sh.at[di_t], add=True)
    plsc.subcore_barrier()

    def _wc(k, _):
        r = rb + k * WCH
        pltpu.sync_copy(acc_sh.at[pl.ds(r, WCH)],
                        out_hbm.at[pl.ds(coff + r, WCH)])
        return 0
    lax.fori_loop(0, nk, _wc, 0)


def _sc_l1(xl_tab, xr_tab, src, dst, att1g):
    mesh = plsc.VectorSubcoreMesh(core_axis_name="c", subcore_axis_name="s",
                                  num_cores=NC, num_subcores=NS)
    f = functools.partial(
        pl.kernel,
        out_type=jax.ShapeDtypeStruct((NC * N, PC1), _f32),
        mesh=mesh,
        compiler_params=pltpu.CompilerParams(
            needs_layout_passes=False, use_tc_tiling_on_sc=False),
        scratch_types=[
            pltpu.VMEM_SHARED((N, PC1), _f32),
            pltpu.VMEM((B, PC1), _f32),
            pltpu.VMEM((B, GC), _f32),
            pltpu.VMEM((B,), _i32),
            pltpu.VMEM((B,), _i32),
            pltpu.VMEM((B,), _i32),
            pltpu.VMEM((B,), _i32),
            pltpu.VMEM((B, PC1), _f32),
            pltpu.VMEM((B, GC), _f32),
            pltpu.VMEM((B,), _i32),
            pltpu.VMEM((B,), _i32),
            pltpu.VMEM((B,), _i32),
            pltpu.VMEM((B,), _i32),
            pltpu.VMEM((T1,), _i32),
            pltpu.VMEM((T1,), _i32),
            pltpu.VMEM((T1,), _i32),
            pltpu.VMEM((GC,), _f32),
            pltpu.SemaphoreType.DMA,
            pltpu.SemaphoreType.DMA,
            pltpu.SemaphoreType.DMA,
            pltpu.SemaphoreType.DMA,
        ],
    )(_sc_l1_body)
    return f(xl_tab, xr_tab, src, dst, att1g)


def _stage_b_body(num_ref, xl_ref, xr_ref, att_ref, b1_ref, wl_ref, wr_ref,
                  xl2_ref, xr2_ref):
    parts = []
    for g in range(2):
        xl = xl_ref[g, :, 0:GC]
        xr = xr_ref[g, :, :]
        sv = xl + xr
        tv = jnp.maximum(sv, 0.2 * sv)
        u = tv * att_ref[g, :][None, :]
        for hh in range(4):
            cs = pl.ds(hh * 32, 32)
            slog = jnp.sum(u[:, hh * 32:(hh + 1) * 32], axis=1)
            wself = jnp.exp(slog)
            num = num_ref[g, :, hh * 32:(hh + 1) * 32] \
                + wself[:, None] * xl[:, hh * 32:(hh + 1) * 32]
            den = num_ref[g, :, GC + hh:GC + hh + 1] + wself[:, None]
            h = num / den + b1_ref[0, (g * 4 + hh) * 32:(g * 4 + hh + 1) * 32][None, :]
            parts.append(h)
    h1 = jnp.concatenate(parts, axis=1)
    h1 = jnp.where(h1 > 0, h1, jnp.exp(jnp.minimum(h1, 0.0)) - 1.0)
    xl2 = jnp.dot(h1, wl_ref[...], preferred_element_type=_f32)
    col = lax.broadcasted_iota(_i32, xl2.shape, 1)
    xl2_ref[...] = jnp.where(col == C2, 1.0, xl2)
    xr2_ref[...] = jnp.dot(h1, wr_ref[...], preferred_element_type=_f32)


def _stage_b(num1, xl1p, xr1p, att1g, b1, W2lp, W2rp):
    bn = 1000
    grid = (N // bn,)
    return pl.pallas_call(
        _stage_b_body,
        grid=grid,
        in_specs=[
            pl.BlockSpec((2, bn, PC1), lambda i: (0, i, 0)),
            pl.BlockSpec((2, bn, PC1), lambda i: (0, i, 0)),
            pl.BlockSpec((2, bn, GC), lambda i: (0, i, 0)),
            pl.BlockSpec((2, GC), lambda i: (0, 0)),
            pl.BlockSpec((1, HC1), lambda i: (0, 0)),
            pl.BlockSpec((HC1, PC2), lambda i: (0, 0)),
            pl.BlockSpec((HC1, PC2), lambda i: (0, 0)),
        ],
        out_specs=[
            pl.BlockSpec((bn, PC2), lambda i: (i, 0)),
            pl.BlockSpec((bn, PC2), lambda i: (i, 0)),
        ],
        out_shape=[
            jax.ShapeDtypeStruct((N, PC2), _f32),
            jax.ShapeDtypeStruct((N, PC2), _f32),
        ],
    )(num1, xl1p, xr1p, att1g, b1, W2lp, W2rp)


def _sc_l2_body(xl_hbm, xr_hbm, src_hbm, dst_hbm, att_hbm, out_hbm,
                acc_sh, xl_v0, xr_v0, gs_v0, di_v0, dc_v0,
                xl_v1, xr_v1, gs_v1, di_v1, dc_v1,
                gs_t, di_t, att_v, gsem0, gsem1, isem0, isem1):
    c = lax.axis_index("c")
    s = lax.axis_index("s")

    pltpu.sync_copy(att_hbm, att_v)

    for i in range(WCH):
        for j in range(PC2 // LN):
            xl_v0[i, pl.ds(j * LN, LN)] = jnp.zeros((LN,), _f32)
    rb = s * RB
    nk = jnp.where(s == NS - 1, 40, 39)

    def _zc(k, _):
        pltpu.sync_copy(xl_v0.at[pl.ds(0, WCH)],
                        acc_sh.at[pl.ds(rb + k * WCH, WCH)])
        return 0
    lax.fori_loop(0, nk, _zc, 0)
    plsc.subcore_barrier()

    ebase = (c * NS + s) * EPT2
    lanes = lax.broadcasted_iota(_i32, (LN,), 0)
    att_regs = [att_v[pl.ds(j * LN, LN)] for j in range(PC2 // LN)]

    def _stage(base, gs_v, di_v, isem):
        pltpu.async_copy(src_hbm.at[pl.ds(base, B)], gs_v, isem)
        pltpu.async_copy(dst_hbm.at[pl.ds(base, B)], di_v, isem)

    def _fire(base, xl_v, xr_v, gs_v, di_v, dc_v, gsem, isem):
        pltpu.make_async_copy(src_hbm.at[pl.ds(base, B)], gs_v, isem).wait()
        pltpu.make_async_copy(dst_hbm.at[pl.ds(base, B)], di_v, isem).wait()
        for i in range(B // LN):
            sl = pl.ds(i * LN, LN)
            dc_v[sl] = di_v[sl]
        pltpu.async_copy(xl_hbm.at[gs_v], xl_v, gsem)
        pltpu.async_copy(xr_hbm.at[di_v], xr_v, gsem)

    def _compute(xl_v, xr_v, gs_v, di_v, dc_v, gsem, stage_fn=None):
        pltpu.make_async_copy(xl_hbm.at[gs_v], xl_v, gsem).wait()
        pltpu.make_async_copy(xr_hbm.at[di_v], xr_v, gsem).wait()
        if stage_fn is not None:
            stage_fn()

        @plsc.parallel_loop(0, B, unroll=8)
        def _erow(e):
            xlr = [xl_v[e, pl.ds(j * LN, LN)] for j in range(PC2 // LN)]
            xrr = [xr_v[e, pl.ds(j * LN, LN)] for j in range(PC2 // LN)]
            u = jnp.zeros((LN,), _f32)
            for j in range(PC2 // LN):
                sv = xlr[j] + xrr[j]
                tv = jnp.maximum(sv, 0.2 * sv)
                u = u + tv * att_regs[j]
            for sh in (1, 2, 4, 8):
                u = u + jnp.take(u, lanes ^ sh)
            w = jnp.exp(u)
            for j in range(PC2 // LN):
                xl_v[e, pl.ds(j * LN, LN)] = xlr[j] * w

        pltpu.sync_copy(xl_v, acc_sh.at[dc_v], add=True)

    nch = EPT2 // B
    npair = nch // 2
    s0 = (xl_v0, xr_v0, gs_v0, di_v0, dc_v0, gsem0, isem0)
    s1 = (xl_v1, xr_v1, gs_v1, di_v1, dc_v1, gsem1, isem1)

    def _idx(slot):
        return (slot[2], slot[3], slot[6])

    def _cmp(slot):
        return (slot[0], slot[1], slot[2], slot[3], slot[4], slot[5])

    _stage(ebase, *_idx(s0))
    _fire(ebase, *s0)
    _stage(ebase + B, *_idx(s1))
    _fire(ebase + B, *s1)

    def _pair(m, _):
        k0 = 2 * m

        def _st0():
            @pl.when(k0 + 2 < nch)
            def _():
                _stage(ebase + (k0 + 2) * B, *_idx(s0))

        def _st1():
            @pl.when(k0 + 3 < nch)
            def _():
                _stage(ebase + (k0 + 3) * B, *_idx(s1))

        _compute(*_cmp(s0), stage_fn=_st0)

        @pl.when(k0 + 2 < nch)
        def _():
            _fire(ebase + (k0 + 2) * B, *s0)
        _compute(*_cmp(s1), stage_fn=_st1)

        @pl.when(k0 + 3 < nch)
        def _():
            _fire(ebase + (k0 + 3) * B, *s1)
        return 0
    lax.fori_loop(0, npair, _pair, 0)

    tb = ebase + nch * B
    pltpu.sync_copy(src_hbm.at[pl.ds(tb, T2)], gs_t)
    pltpu.sync_copy(dst_hbm.at[pl.ds(tb, T2)], di_t)
    xl_t = xl_v0.at[pl.ds(0, T2)]
    xr_t = xr_v0.at[pl.ds(0, T2)]
    cp1 = pltpu.async_copy(xl_hbm.at[gs_t], xl_t, gsem0)
    cp2 = pltpu.async_copy(xr_hbm.at[di_t], xr_t, gsem0)
    cp1.wait()
    cp2.wait()

    @plsc.parallel_loop(0, T2, unroll=8)
    def _erow_t(e):
        xlr = [xl_v0[e, pl.ds(j * LN, LN)] for j in range(PC2 // LN)]
        xrr = [xr_v0[e, pl.ds(j * LN, LN)] for j in range(PC2 // LN)]
        u = jnp.zeros((LN,), _f32)
        for j in range(PC2 // LN):
            sv = xlr[j] + xrr[j]
            tv = jnp.maximum(sv, 0.2 * sv)
            u = u + tv * att_regs[j]
        for sh in (1, 2, 4, 8):
            u = u + jnp.take(u, lanes ^ sh)
        w = jnp.exp(u)
        for j in range(PC2 // LN):
            xl_v0[e, pl.ds(j * LN, LN)] = xlr[j] * w

    pltpu.sync_copy(xl_t, acc_sh.at[di_t], add=True)
    plsc.subcore_barrier()

    def _wc(k, _):
        r = rb + k * WCH
        pltpu.sync_copy(acc_sh.at[pl.ds(r, WCH)],
                        out_hbm.at[pl.ds(c * N + r, WCH)])
        return 0
    lax.fori_loop(0, nk, _wc, 0)


def _sc_l2(xl2p, xr2p, src, dst, att2p):
    mesh = plsc.VectorSubcoreMesh(core_axis_name="c", subcore_axis_name="s",
                                  num_cores=NC, num_subcores=NS)
    f = functools.partial(
        pl.kernel,
        out_type=jax.ShapeDtypeStruct((NC * N, PC2), _f32),
        mesh=mesh,
        compiler_params=pltpu.CompilerParams(
            needs_layout_passes=False, use_tc_tiling_on_sc=False),
        scratch_types=[
            pltpu.VMEM_SHARED((N, PC2), _f32),
            pltpu.VMEM((B, PC2), _f32),
            pltpu.VMEM((B, PC2), _f32),
            pltpu.VMEM((B,), _i32),
            pltpu.VMEM((B,), _i32),
            pltpu.VMEM((B,), _i32),
            pltpu.VMEM((B, PC2), _f32),
            pltpu.VMEM((B, PC2), _f32),
            pltpu.VMEM((B,), _i32),
            pltpu.VMEM((B,), _i32),
            pltpu.VMEM((B,), _i32),
            pltpu.VMEM((T2,), _i32),
            pltpu.VMEM((T2,), _i32),
            pltpu.VMEM((PC2,), _f32),
            pltpu.SemaphoreType.DMA,
            pltpu.SemaphoreType.DMA,
            pltpu.SemaphoreType.DMA,
            pltpu.SemaphoreType.DMA,
        ],
    )(_sc_l2_body)
    return f(xl2p, xr2p, src, dst, att2p)


def _stage_c_body(num_ref, xl_ref, xr_ref, att_ref, b2_ref, out_ref):
    xl = xl_ref[...]
    sv = xl + xr_ref[...]
    tv = jnp.maximum(sv, 0.2 * sv)
    u = tv * att_ref[0, :][None, :]
    slog = jnp.sum(u, axis=1)
    wself = jnp.exp(slog)
    numt = num_ref[0] + num_ref[1] + wself[:, None] * xl
    den = numt[:, C2:C2 + 1]
    out = numt[:, 0:C2] / den + b2_ref[0, 0:C2][None, :]
    out_ref[...] = jnp.maximum(out, 0.01 * out)


def _stage_c(num2, xl2p, xr2p, att2p, b2p):
    bn = 1000
    grid = (N // bn,)
    return pl.pallas_call(
        _stage_c_body,
        grid=grid,
        in_specs=[
            pl.BlockSpec((2, bn, PC2), lambda i: (0, i, 0)),
            pl.BlockSpec((bn, PC2), lambda i: (i, 0)),
            pl.BlockSpec((bn, PC2), lambda i: (i, 0)),
            pl.BlockSpec((1, PC2), lambda i: (0, 0)),
            pl.BlockSpec((1, PC2), lambda i: (0, 0)),
        ],
        out_specs=pl.BlockSpec((bn, C2), lambda i: (i, 0)),
        out_shape=jax.ShapeDtypeStruct((N, C2), _f32),
    )(num2, xl2p, xr2p, att2p, b2p)


def kernel(x, edge_index, W1l, W1r, att1, b1, W2l, W2r, att2, b2):
    src = edge_index[0].astype(_i32)
    dst = edge_index[1].astype(_i32)

    xl1p, xr1p = _stage_a(x, W1l, W1r)
    att1g = att1.reshape(2, GC)
    num1 = _sc_l1(xl1p.reshape(NC * N, PC1), xr1p.reshape(NC * N, GC),
                  src, dst, att1g)

    W2lp = jnp.pad(W2l, ((0, 0), (0, PC2 - C2)))
    W2rp = jnp.pad(W2r, ((0, 0), (0, PC2 - C2)))
    xl2p, xr2p = _stage_b(num1.reshape(2, N, PC1), xl1p, xr1p, att1g,
                          b1.reshape(1, HC1), W2lp, W2rp)

    att2p = jnp.pad(att2.reshape(C2), (0, PC2 - C2))
    num2 = _sc_l2(xl2p, xr2p, src, dst, att2p)

    b2p = jnp.pad(b2, (0, PC2 - C2)).reshape(1, PC2)
    return _stage_c(num2.reshape(2, N, PC2), xl2p, xr2p,
                    att2p.reshape(1, PC2), b2p)


# speedup vs baseline: 1.1258x; 1.1258x over previous
import functools

import jax
import jax.numpy as jnp
from jax import lax
from jax.experimental import pallas as pl
from jax.experimental.pallas import tpu as pltpu
from jax.experimental.pallas import tpu_sc as plsc

N = 10000
E = 320000
D = 128
HC1 = 256
GC = 128
PC1 = 144
C2 = 40
PC2 = 48
NC = 2
NS = 16
LN = 16
B = 64
T1 = 32
T2 = 16
EPT1 = E // NS
EPT2 = E // (NC * NS)
RB = 624
WCH = 16

_f32 = jnp.float32
_i32 = jnp.int32


def _stage_a_body(x_ref, wl_ref, wr_ref, xl_ref, xr_ref):
    xb = x_ref[...]
    ml = jnp.dot(xb, wl_ref[...], preferred_element_type=_f32)
    mr = jnp.dot(xb, wr_ref[...], preferred_element_type=_f32)
    xl_ref[0, :, 0:GC] = ml
    col = lax.broadcasted_iota(_i32, (ml.shape[0], PC1 - GC), 1)
    xl_ref[0, :, GC:PC1] = jnp.where(col < 4, 1.0, 0.0).astype(_f32)
    xr_ref[0, :, :] = mr


def _stage_a(x, W1l, W1r):
    bn = 1000
    grid = (N // bn, 2)
    return pl.pallas_call(
        _stage_a_body,
        grid=grid,
        in_specs=[
            pl.BlockSpec((bn, D), lambda i, g: (i, 0)),
            pl.BlockSpec((D, GC), lambda i, g: (0, g)),
            pl.BlockSpec((D, GC), lambda i, g: (0, g)),
        ],
        out_specs=[
            pl.BlockSpec((1, bn, PC1), lambda i, g: (g, i, 0)),
            pl.BlockSpec((1, bn, GC), lambda i, g: (g, i, 0)),
        ],
        out_shape=[
            jax.ShapeDtypeStruct((2, N, PC1), _f32),
            jax.ShapeDtypeStruct((2, N, GC), _f32),
        ],
    )(x, W1l, W1r)


def _sc_l1_body(xl_hbm, xr_hbm, src_hbm, dst_hbm, att_hbm, out_hbm,
                acc_sh, xl_v0, xr_v0, gs_v0, gd_v0, di_v0, dc_v0,
                xl_v1, xr_v1, gs_v1, gd_v1, di_v1, dc_v1,
                gs_t, gd_t, di_t, att_v, gsem0, gsem1, isem0, isem1):
    c = lax.axis_index("c")
    s = lax.axis_index("s")

    pltpu.sync_copy(att_hbm.at[c], att_v)

    for i in range(WCH):
        for j in range(PC1 // LN):
            xl_v0[i, pl.ds(j * LN, LN)] = jnp.zeros((LN,), _f32)
    rb = s * RB
    nk = jnp.where(s == NS - 1, 40, 39)

    def _zc(k, _):
        pltpu.sync_copy(xl_v0.at[pl.ds(0, WCH)],
                        acc_sh.at[pl.ds(rb + k * WCH, WCH)])
        return 0
    lax.fori_loop(0, nk, _zc, 0)
    plsc.subcore_barrier()

    ebase = s * EPT1
    lanes = lax.broadcasted_iota(_i32, (LN,), 0)
    coff = c * N
    att_regs = [att_v[pl.ds(j * LN, LN)] for j in range(GC // LN)]

    def _stage(base, gs_v, di_v, isem):
        pltpu.async_copy(src_hbm.at[pl.ds(base, B)], gs_v, isem)
        pltpu.async_copy(dst_hbm.at[pl.ds(base, B)], di_v, isem)

    def _fire(base, xl_v, xr_v, gs_v, gd_v, di_v, dc_v, gsem, isem):
        pltpu.make_async_copy(src_hbm.at[pl.ds(base, B)], gs_v, isem).wait()
        pltpu.make_async_copy(dst_hbm.at[pl.ds(base, B)], di_v, isem).wait()
        for i in range(B // LN):
            sl = pl.ds(i * LN, LN)
            d = di_v[sl]
            gs_v[sl] = gs_v[sl] + coff
            gd_v[sl] = d + coff
            dc_v[sl] = d
        pltpu.async_copy(xl_hbm.at[gs_v], xl_v, gsem)
        pltpu.async_copy(xr_hbm.at[gd_v], xr_v, gsem)

    def _compute(xl_v, xr_v, gs_v, gd_v, dc_v, gsem, stage_fn=None):
        pltpu.make_async_copy(xl_hbm.at[gs_v], xl_v, gsem).wait()
        pltpu.make_async_copy(xr_hbm.at[gd_v], xr_v, gsem).wait()
        if stage_fn is not None:
            stage_fn()

        @plsc.parallel_loop(0, B, unroll=2)
        def _erow(e):
            xlr = [xl_v[e, pl.ds(j * LN, LN)] for j in range(GC // LN)]
            xrr = [xr_v[e, pl.ds(j * LN, LN)] for j in range(GC // LN)]
            ws = []
            for h in range(4):
                u = jnp.zeros((LN,), _f32)
                for j in (2 * h, 2 * h + 1):
                    sv = xlr[j] + xrr[j]
                    tv = jnp.maximum(sv, 0.2 * sv)
                    u = u + tv * att_regs[j]
                for sh in (1, 2, 4, 8):
                    u = u + jnp.take(u, lanes ^ sh)
                ws.append(jnp.exp(u))
            for j in range(GC // LN):
                xl_v[e, pl.ds(j * LN, LN)] = xlr[j] * ws[j // 2]
            wp = jnp.where(lanes == 0, ws[0],
                           jnp.where(lanes == 1, ws[1],
                                     jnp.where(lanes == 2, ws[2],
                                               jnp.where(lanes == 3, ws[3],
                                                         0.0))))
            xl_v[e, pl.ds(GC, LN)] = wp

        pltpu.sync_copy(xl_v, acc_sh.at[dc_v], add=True)

    nch = EPT1 // B
    npair = nch // 2
    s0 = (xl_v0, xr_v0, gs_v0, gd_v0, di_v0, dc_v0, gsem0, isem0)
    s1 = (xl_v1, xr_v1, gs_v1, gd_v1, di_v1, dc_v1, gsem1, isem1)

    def _idx(slot):
        return (slot[2], slot[4], slot[7])

    def _cmp(slot):
        return (slot[0], slot[1], slot[2], slot[3], slot[5], slot[6])

    _stage(ebase, *_idx(s0))
    _fire(ebase, *s0)
    _stage(ebase + B, *_idx(s1))
    _fire(ebase + B, *s1)

    def _pair(m, _):
        k0 = 2 * m

        def _st0():
            @pl.when(k0 + 2 < nch)
            def _():
                _stage(ebase + (k0 + 2) * B, *_idx(s0))

        def _st1():
            @pl.when(k0 + 3 < nch)
            def _():
                _stage(ebase + (k0 + 3) * B, *_idx(s1))

        _compute(*_cmp(s0), stage_fn=_st0)

        @pl.when(k0 + 2 < nch)
        def _():
            _fire(ebase + (k0 + 2) * B, *s0)
        _compute(*_cmp(s1), stage_fn=_st1)

        @pl.when(k0 + 3 < nch)
        def _():
            _fire(ebase + (k0 + 3) * B, *s1)
        return 0
    lax.fori_loop(0, npair, _pair, 0)

    tb = ebase + nch * B
    pltpu.sync_copy(src_hbm.at[pl.ds(tb, T1)], gs_t)
    pltpu.sync_copy(dst_hbm.at[pl.ds(tb, T1)], di_t)
    for i in range(T1 // LN):
        sl = pl.ds(i * LN, LN)
        gs_t[sl] = gs_t[sl] + coff
        gd_t[sl] = di_t[sl] + coff
    xl_t = xl_v0.at[pl.ds(0, T1)]
    xr_t = xr_v0.at[pl.ds(0, T1)]
    cp1 = pltpu.async_copy(xl_hbm.at[gs_t], xl_t, gsem0)
    cp2 = pltpu.async_copy(xr_hbm.at[gd_t], xr_t, gsem0)
    cp1.wait()
    cp2.wait()

    @plsc.parallel_loop(0, T1, unroll=2)
    def _erow_t(e):
        xlr = [xl_v0[e, pl.ds(j * LN, LN)] for j in range(GC // LN)]
        xrr = [xr_v0[e, pl.ds(j * LN, LN)] for j in range(GC // LN)]
        ws = []
        for h in range(4):
            u = jnp.zeros((LN,), _f32)
            for j in (2 * h, 2 * h + 1):
                sv = xlr[j] + xrr[j]
                tv = jnp.maximum(sv, 0.2 * sv)
                u = u + tv * att_regs[j]
            for sh in (1, 2, 4, 8):
                u = u + jnp.take(u, lanes ^ sh)
            ws.append(jnp.exp(u))
        for j in range(GC // LN):
            xl_v0[e, pl.ds(j * LN, LN)] = xlr[j] * ws[j // 2]
        wp = jnp.where(lanes == 0, ws[0],
                       jnp.where(lanes == 1, ws[1],
                                 jnp.where(lanes == 2, ws[2],
                                           jnp.where(lanes == 3, ws[3],
                                                     0.0))))
        xl_v0[e, pl.ds(GC, LN)] = wp

    pltpu.sync_copy(xl_t, acc_sh.at[di_t], add=True)
    plsc.subcore_barrier()

    def _wc(k, _):
        r = rb + k * WCH
        pltpu.sync_copy(acc_sh.at[pl.ds(r, WCH)],
                        out_hbm.at[pl.ds(coff + r, WCH)])
        return 0
    lax.fori_loop(0, nk, _wc, 0)


def _sc_l1(xl_tab, xr_tab, src, dst, att1g):
    mesh = plsc.VectorSubcoreMesh(core_axis_name="c", subcore_axis_name="s",
                                  num_cores=NC, num_subcores=NS)
    f = functools.partial(
        pl.kernel,
        out_type=jax.ShapeDtypeStruct((NC * N, PC1), _f32),
        mesh=mesh,
        compiler_params=pltpu.CompilerParams(
            needs_layout_passes=False, use_tc_tiling_on_sc=False),
        scratch_types=[
            pltpu.VMEM_SHARED((N, PC1), _f32),
            pltpu.VMEM((B, PC1), _f32),
            pltpu.VMEM((B, GC), _f32),
            pltpu.VMEM((B,), _i32),
            pltpu.VMEM((B,), _i32),
            pltpu.VMEM((B,), _i32),
            pltpu.VMEM((B,), _i32),
            pltpu.VMEM((B, PC1), _f32),
            pltpu.VMEM((B, GC), _f32),
            pltpu.VMEM((B,), _i32),
            pltpu.VMEM((B,), _i32),
            pltpu.VMEM((B,), _i32),
            pltpu.VMEM((B,), _i32),
            pltpu.VMEM((T1,), _i32),
            pltpu.VMEM((T1,), _i32),
            pltpu.VMEM((T1,), _i32),
            pltpu.VMEM((GC,), _f32),
            pltpu.SemaphoreType.DMA,
            pltpu.SemaphoreType.DMA,
            pltpu.SemaphoreType.DMA,
            pltpu.SemaphoreType.DMA,
        ],
    )(_sc_l1_body)
    return f(xl_tab, xr_tab, src, dst, att1g)


def _stage_b_body(num_ref, xl_ref, xr_ref, att_ref, b1_ref, wl_ref, wr_ref,
                  xl2_ref, xr2_ref):
    parts = []
    for g in range(2):
        xl = xl_ref[g, :, 0:GC]
        xr = xr_ref[g, :, :]
        sv = xl + xr
        tv = jnp.maximum(sv, 0.2 * sv)
        u = tv * att_ref[g, :][None, :]
        for hh in range(4):
            cs = pl.ds(hh * 32, 32)
            slog = jnp.sum(u[:, hh * 32:(hh + 1) * 32], axis=1)
            wself = jnp.exp(slog)
            num = num_ref[g, :, hh * 32:(hh + 1) * 32] \
                + wself[:, None] * xl[:, hh * 32:(hh + 1) * 32]
            den = num_ref[g, :, GC + hh:GC + hh + 1] + wself[:, None]
            h = num / den + b1_ref[0, (g * 4 + hh) * 32:(g * 4 + hh + 1) * 32][None, :]
            parts.append(h)
    h1 = jnp.concatenate(parts, axis=1)
    h1 = jnp.where(h1 > 0, h1, jnp.exp(jnp.minimum(h1, 0.0)) - 1.0)
    xl2 = jnp.dot(h1, wl_ref[...], preferred_element_type=_f32)
    col = lax.broadcasted_iota(_i32, xl2.shape, 1)
    xl2_ref[...] = jnp.where(col == C2, 1.0, xl2)
    xr2_ref[...] = jnp.dot(h1, wr_ref[...], preferred_element_type=_f32)


def _stage_b(num1, xl1p, xr1p, att1g, b1, W2lp, W2rp):
    bn = 1000
    grid = (N // bn,)
    return pl.pallas_call(
        _stage_b_body,
        grid=grid,
        in_specs=[
            pl.BlockSpec((2, bn, PC1), lambda i: (0, i, 0)),
            pl.BlockSpec((2, bn, PC1), lambda i: (0, i, 0)),
            pl.BlockSpec((2, bn, GC), lambda i: (0, i, 0)),
            pl.BlockSpec((2, GC), lambda i: (0, 0)),
            pl.BlockSpec((1, HC1), lambda i: (0, 0)),
            pl.BlockSpec((HC1, PC2), lambda i: (0, 0)),
            pl.BlockSpec((HC1, PC2), lambda i: (0, 0)),
        ],
        out_specs=[
            pl.BlockSpec((bn, PC2), lambda i: (i, 0)),
            pl.BlockSpec((bn, PC2), lambda i: (i, 0)),
        ],
        out_shape=[
            jax.ShapeDtypeStruct((N, PC2), _f32),
            jax.ShapeDtypeStruct((N, PC2), _f32),
        ],
    )(num1, xl1p, xr1p, att1g, b1, W2lp, W2rp)


def _sc_l2_body(xl_hbm, xr_hbm, src_hbm, dst_hbm, att_hbm, out_hbm,
                acc_sh, xl_v0, xr_v0, gs_v0, di_v0, dc_v0,
                xl_v1, xr_v1, gs_v1, di_v1, dc_v1,
                gs_t, di_t, att_v, gsem0, gsem1, isem0, isem1):
    c = lax.axis_index("c")
    s = lax.axis_index("s")

    pltpu.sync_copy(att_hbm, att_v)

    for i in range(WCH):
        for j in range(PC2 // LN):
            xl_v0[i, pl.ds(j * LN, LN)] = jnp.zeros((LN,), _f32)
    rb = s * RB
    nk = jnp.where(s == NS - 1, 40, 39)

    def _zc(k, _):
        pltpu.sync_copy(xl_v0.at[pl.ds(0, WCH)],
                        acc_sh.at[pl.ds(rb + k * WCH, WCH)])
        return 0
    lax.fori_loop(0, nk, _zc, 0)
    plsc.subcore_barrier()

    ebase = (c * NS + s) * EPT2
    lanes = lax.broadcasted_iota(_i32, (LN,), 0)
    att_regs = [att_v[pl.ds(j * LN, LN)] for j in range(PC2 // LN)]

    def _stage(base, gs_v, di_v, isem):
        pltpu.async_copy(src_hbm.at[pl.ds(base, B)], gs_v, isem)
        pltpu.async_copy(dst_hbm.at[pl.ds(base, B)], di_v, isem)

    def _fire(base, xl_v, xr_v, gs_v, di_v, dc_v, gsem, isem):
        pltpu.make_async_copy(src_hbm.at[pl.ds(base, B)], gs_v, isem).wait()
        pltpu.make_async_copy(dst_hbm.at[pl.ds(base, B)], di_v, isem).wait()
        for i in range(B // LN):
            sl = pl.ds(i * LN, LN)
            dc_v[sl] = di_v[sl]
        pltpu.async_copy(xl_hbm.at[gs_v], xl_v, gsem)
        pltpu.async_copy(xr_hbm.at[di_v], xr_v, gsem)

    def _compute(xl_v, xr_v, gs_v, di_v, dc_v, gsem, stage_fn=None):
        pltpu.make_async_copy(xl_hbm.at[gs_v], xl_v, gsem).wait()
        pltpu.make_async_copy(xr_hbm.at[di_v], xr_v, gsem).wait()
        if stage_fn is not None:
            stage_fn()

        @plsc.parallel_loop(0, B, unroll=8)
        def _erow(e):
            xlr = [xl_v[e, pl.ds(j * LN, LN)] for j in range(PC2 // LN)]
            xrr = [xr_v[e, pl.ds(j * LN, LN)] for j in range(PC2 // LN)]
            u = jnp.zeros((LN,), _f32)
            for j in range(PC2 // LN):
                sv = xlr[j] + xrr[j]
                tv = jnp.maximum(sv, 0.2 * sv)
                u = u + tv * att_regs[j]
            for sh in (1, 2, 4, 8):
                u = u + jnp.take(u, lanes ^ sh)
            w = jnp.exp(u)
            for j in range(PC2 // LN):
                xl_v[e, pl.ds(j * LN, LN)] = xlr[j] * w

        pltpu.sync_copy(xl_v, acc_sh.at[dc_v], add=True)

    nch = EPT2 // B
    npair = nch // 2
    s0 = (xl_v0, xr_v0, gs_v0, di_v0, dc_v0, gsem0, isem0)
    s1 = (xl_v1, xr_v1, gs_v1, di_v1, dc_v1, gsem1, isem1)

    def _idx(slot):
        return (slot[2], slot[3], slot[6])

    def _cmp(slot):
        return (slot[0], slot[1], slot[2], slot[3], slot[4], slot[5])

    _stage(ebase, *_idx(s0))
    _fire(ebase, *s0)
    _stage(ebase + B, *_idx(s1))
    _fire(ebase + B, *s1)

    def _pair(m, _):
        k0 = 2 * m

        def _st0():
            @pl.when(k0 + 2 < nch)
            def _():
                _stage(ebase + (k0 + 2) * B, *_idx(s0))

        def _st1():
            @pl.when(k0 + 3 < nch)
            def _():
                _stage(ebase + (k0 + 3) * B, *_idx(s1))

        _compute(*_cmp(s0), stage_fn=_st0)

        @pl.when(k0 + 2 < nch)
        def _():
            _fire(ebase + (k0 + 2) * B, *s0)
        _compute(*_cmp(s1), stage_fn=_st1)

        @pl.when(k0 + 3 < nch)
        def _():
            _fire(ebase + (k0 + 3) * B, *s1)
        return 0
    lax.fori_loop(0, npair, _pair, 0)

    tb = ebase + nch * B
    pltpu.sync_copy(src_hbm.at[pl.ds(tb, T2)], gs_t)
    pltpu.sync_copy(dst_hbm.at[pl.ds(tb, T2)], di_t)
    xl_t = xl_v0.at[pl.ds(0, T2)]
    xr_t = xr_v0.at[pl.ds(0, T2)]
    cp1 = pltpu.async_copy(xl_hbm.at[gs_t], xl_t, gsem0)
    cp2 = pltpu.async_copy(xr_hbm.at[di_t], xr_t, gsem0)
    cp1.wait()
    cp2.wait()

    @plsc.parallel_loop(0, T2, unroll=8)
    def _erow_t(e):
        xlr = [xl_v0[e, pl.ds(j * LN, LN)] for j in range(PC2 // LN)]
        xrr = [xr_v0[e, pl.ds(j * LN, LN)] for j in range(PC2 // LN)]
        u = jnp.zeros((LN,), _f32)
        for j in range(PC2 // LN):
            sv = xlr[j] + xrr[j]
            tv = jnp.maximum(sv, 0.2 * sv)
            u = u + tv * att_regs[j]
        for sh in (1, 2, 4, 8):
            u = u + jnp.take(u, lanes ^ sh)
        w = jnp.exp(u)
        for j in range(PC2 // LN):
            xl_v0[e, pl.ds(j * LN, LN)] = xlr[j] * w

    pltpu.sync_copy(xl_t, acc_sh.at[di_t], add=True)
    plsc.subcore_barrier()

    def _wc(k, _):
        r = rb + k * WCH
        pltpu.sync_copy(acc_sh.at[pl.ds(r, WCH)],
                        out_hbm.at[pl.ds(c * N + r, WCH)])
        return 0
    lax.fori_loop(0, nk, _wc, 0)


def _sc_l2(xl2p, xr2p, src, dst, att2p):
    mesh = plsc.VectorSubcoreMesh(core_axis_name="c", subcore_axis_name="s",
                                  num_cores=NC, num_subcores=NS)
    f = functools.partial(
        pl.kernel,
        out_type=jax.ShapeDtypeStruct((NC * N, PC2), _f32),
        mesh=mesh,
        compiler_params=pltpu.CompilerParams(
            needs_layout_passes=False, use_tc_tiling_on_sc=False),
        scratch_types=[
            pltpu.VMEM_SHARED((N, PC2), _f32),
            pltpu.VMEM((B, PC2), _f32),
            pltpu.VMEM((B, PC2), _f32),
            pltpu.VMEM((B,), _i32),
            pltpu.VMEM((B,), _i32),
            pltpu.VMEM((B,), _i32),
            pltpu.VMEM((B, PC2), _f32),
            pltpu.VMEM((B, PC2), _f32),
            pltpu.VMEM((B,), _i32),
            pltpu.VMEM((B,), _i32),
            pltpu.VMEM((B,), _i32),
            pltpu.VMEM((T2,), _i32),
            pltpu.VMEM((T2,), _i32),
            pltpu.VMEM((PC2,), _f32),
            pltpu.SemaphoreType.DMA,
            pltpu.SemaphoreType.DMA,
            pltpu.SemaphoreType.DMA,
            pltpu.SemaphoreType.DMA,
        ],
    )(_sc_l2_body)
    return f(xl2p, xr2p, src, dst, att2p)


def _stage_c_body(num_ref, xl_ref, xr_ref, att_ref, b2_ref, out_ref):
    xl = xl_ref[...]
    sv = xl + xr_ref[...]
    tv = jnp.maximum(sv, 0.2 * sv)
    u = tv * att_ref[0, :][None, :]
    slog = jnp.sum(u, axis=1)
    wself = jnp.exp(slog)
    numt = num_ref[0] + num_ref[1] + wself[:, None] * xl
    den = numt[:, C2:C2 + 1]
    out = numt[:, 0:C2] / den + b2_ref[0, 0:C2][None, :]
    out_ref[...] = jnp.maximum(out, 0.01 * out)


def _stage_c(num2, xl2p, xr2p, att2p, b2p):
    bn = 1000
    grid = (N // bn,)
    return pl.pallas_call(
        _stage_c_body,
        grid=grid,
        in_specs=[
            pl.BlockSpec((2, bn, PC2), lambda i: (0, i, 0)),
            pl.BlockSpec((bn, PC2), lambda i: (i, 0)),
            pl.BlockSpec((bn, PC2), lambda i: (i, 0)),
            pl.BlockSpec((1, PC2), lambda i: (0, 0)),
            pl.BlockSpec((1, PC2), lambda i: (0, 0)),
        ],
        out_specs=pl.BlockSpec((bn, C2), lambda i: (i, 0)),
        out_shape=jax.ShapeDtypeStruct((N, C2), _f32),
    )(num2, xl2p, xr2p, att2p, b2p)


def kernel(x, edge_index, W1l, W1r, att1, b1, W2l, W2r, att2, b2):
    src = edge_index[0].astype(_i32)
    dst = edge_index[1].astype(_i32)

    xl1p, xr1p = _stage_a(x, W1l, W1r)
    att1g = att1.reshape(2, GC)
    num1 = _sc_l1(xl1p.reshape(NC * N, PC1), xr1p.reshape(NC * N, GC),
                  src, dst, att1g)

    W2lp = jnp.pad(W2l, ((0, 0), (0, PC2 - C2)))
    W2rp = jnp.pad(W2r, ((0, 0), (0, PC2 - C2)))
    xl2p, xr2p = _stage_b(num1.reshape(2, N, PC1), xl1p, xr1p, att1g,
                          b1.reshape(1, HC1), W2lp, W2rp)

    att2p = jnp.pad(att2.reshape(C2), (0, PC2 - C2))
    num2 = _sc_l2(xl2p, xr2p, src, dst, att2p)

    b2p = jnp.pad(b2, (0, PC2 - C2)).reshape(1, PC2)
    return _stage_c(num2.reshape(2, N, PC2), xl2p, xr2p,
                    att2p.reshape(1, PC2), b2p)
